# baseline (speedup 1.0000x reference)
import functools

import jax
import jax.numpy as jnp
from jax.experimental import pallas as pl
from jax.experimental.pallas import tpu as pltpu

N = 2048
D = 768
H = 1536
E = 8
EMBED = 768
TB = 256


def _fused_body(x_ref, wg_ref, W1_ref, b1_ref, W2_ref, b2_ref, pw_ref, pb_ref,
                out_ref, acc_ref, gates_ref):
    e = pl.program_id(1)

    @pl.when(e == 0)
    def _route():
        xb = x_ref[...]
        logits = jnp.dot(xb, wg_ref[...])
        iota = jax.lax.broadcasted_iota(jnp.int32, (TB, E), 1)
        m1 = jnp.max(logits, axis=1, keepdims=True)
        a1 = jnp.min(jnp.where(logits == m1, iota, E), axis=1, keepdims=True)
        masked = jnp.where(iota == a1, -jnp.inf, logits)
        m2 = jnp.max(masked, axis=1, keepdims=True)
        a2 = jnp.min(jnp.where(masked == m2, iota, E), axis=1, keepdims=True)
        t = jnp.exp(m2 - m1)
        g1 = 1.0 / (1.0 + t)
        g2 = t / (1.0 + t)
        gates_ref[...] = (jnp.where(iota == a1, g1, 0.0)
                          + jnp.where(iota == a2, g2, 0.0))
        acc_ref[...] = jnp.zeros_like(acc_ref)

    xb = x_ref[...]
    h = jax.nn.gelu(jnp.dot(xb, W1_ref[0]) + b1_ref[0])
    y = jnp.dot(h, W2_ref[0]) + b2_ref[0]
    lane = jax.lax.broadcasted_iota(jnp.int32, (TB, E), 1)
    g = jnp.sum(jnp.where(lane == e, gates_ref[...], 0.0),
                axis=1, keepdims=True)
    acc_ref[...] += g * y

    @pl.when(e == E - 1)
    def _proj():
        out_ref[...] = jax.lax.dot_general(
            acc_ref[...], pw_ref[...],
            (((1,), (1,)), ((), ()))) + pb_ref[...]


def kernel(x, w_gate, W1, b1, W2, b2, proj_w, proj_b):
    grid = (N // TB, E)
    out = pl.pallas_call(
        _fused_body,
        grid=grid,
        in_specs=[
            pl.BlockSpec((TB, D), lambda t, e: (t, 0)),
            pl.BlockSpec((D, E), lambda t, e: (0, 0)),
            pl.BlockSpec((1, D, H), lambda t, e: (e, 0, 0)),
            pl.BlockSpec((1, 1, H), lambda t, e: (e, 0, 0)),
            pl.BlockSpec((1, H, D), lambda t, e: (e, 0, 0)),
            pl.BlockSpec((1, 1, D), lambda t, e: (e, 0, 0)),
            pl.BlockSpec((EMBED, D), lambda t, e: (0, 0)),
            pl.BlockSpec((1, EMBED), lambda t, e: (0, 0)),
        ],
        out_specs=pl.BlockSpec((TB, EMBED), lambda t, e: (t, 0)),
        out_shape=jax.ShapeDtypeStruct((N, EMBED), jnp.float32),
        scratch_shapes=[
            pltpu.VMEM((TB, D), jnp.float32),
            pltpu.VMEM((TB, E), jnp.float32),
        ],
        compiler_params=pltpu.CompilerParams(
            dimension_semantics=("parallel", "arbitrary")),
    )(x, w_gate, W1, b1.reshape(E, 1, H), W2, b2.reshape(E, 1, D),
      proj_w, proj_b.reshape(1, EMBED))
    return out
